# baseline (speedup 1.0000x reference)
import functools

import jax
import jax.numpy as jnp
from jax import lax
from jax.experimental import pallas as pl
from jax.experimental.pallas import tpu as pltpu
from jax.experimental.pallas import tpu_sc as plsc

N_NODES = 100000
N_SAMPLES = 10
N_EDGES = 1600000
W = 16
N_PAD = 100352
NC, NS = 2, 16
NW = NC * NS
CHUNK = 128
KC = 4
N_CHUNKS = N_EDGES // CHUNK
ROWS_PER_TILE = N_PAD // NS
ZROWS = 784
NBC = 2048


def _prep_body(z_ref, dg_ref, pd_ref, mn_ref, o_ref):
    z = z_ref[...]
    std = jax.nn.softplus(dg_ref[...])
    x = std * z
    row = lax.broadcasted_iota(jnp.int32, (W, NBC), 0)
    x = jnp.where(row == 10, 1.0, x)
    x = jnp.where(row == 11, jax.nn.softplus(pd_ref[...]), x)
    x = jnp.where(row == 12, mn_ref[...], x)
    o_ref[...] = x


RB = 256


def _final_body(s_ref, xs_ref, a_ref, o_ref):
    alpha1 = s_ref[0, 0]
    alpha2 = s_ref[0, 1]
    gamma = s_ref[0, 2]
    xs = xs_ref[...]
    a = a_ref[...]
    aggr = a[0] + a[1]
    r_i = lax.broadcasted_iota(jnp.int32, (128, 128), 0)
    c_i = lax.broadcasted_iota(jnp.int32, (128, 128), 1)
    grp = (c_i // W) * W
    b10 = (r_i == grp + 10).astype(jnp.float32)
    b11 = (r_i == grp + 11).astype(jnp.float32)
    b12 = (r_i == grp + 12).astype(jnp.float32)
    deg = jnp.maximum(jnp.dot(aggr, b10, preferred_element_type=jnp.float32,
                              precision=lax.Precision.HIGHEST),
                      1.0)
    pd = jnp.dot(xs, b11, preferred_element_type=jnp.float32,
                 precision=lax.Precision.HIGHEST)
    mn = jnp.dot(xs, b12, preferred_element_type=jnp.float32,
                 precision=lax.Precision.HIGHEST)
    self_w = jnp.exp(gamma * jnp.log(deg))
    out = pd * (alpha1 * self_w * xs + alpha2 * aggr) + mn
    o_ref[...] = out


def _sc_body(xs_hbm, ei_hbm, zeros_hbm, out_hbm,
             src_v, dst_v, rows_v, acc_sh, gsem, ssem, dsem):
    c = lax.axis_index("c")
    s = lax.axis_index("s")
    base_blk = jnp.where(
        c == 0,
        jnp.where(s < 10, s * 99, 990 + (s - 10) * 98),
        1578 + jnp.where(s < 11, s * 97, 1067 + (s - 11) * 96))
    base_chunk = base_blk * KC
    nblk = jnp.where(c == 0, jnp.where(s < 10, 99, 98),
                     jnp.where(s < 11, 97, 96))

    for r in range(ROWS_PER_TILE // ZROWS):
        pltpu.sync_copy(zeros_hbm,
                        acc_sh.at[pl.ds(s * ROWS_PER_TILE + r * ZROWS, ZROWS)])
    plsc.subcore_barrier()

    def gather(k):
        return pltpu.async_copy(xs_hbm.at[src_v.at[pl.ds(k * CHUNK, CHUNK)]],
                                rows_v.at[k], gsem)

    def scatter(k):
        return pltpu.async_copy(rows_v.at[k], acc_sh.at[dst_v.at[k]], ssem,
                                add=True)

    def body(blk, carry):
        e0 = (base_chunk + blk * KC) * CHUNK
        pltpu.sync_copy(ei_hbm.at[0, pl.ds(e0, KC * CHUNK)], src_v)
        for k in range(KC):
            pltpu.async_copy(ei_hbm.at[1, pl.ds(e0 + k * CHUNK, CHUNK)],
                             dst_v.at[k], dsem)
        for k in range(KC):
            pltpu.make_async_copy(ei_hbm.at[1, pl.ds(e0 + k * CHUNK, CHUNK)],
                                  dst_v.at[k], dsem).wait()
        gather(0)
        for k in range(1, KC):
            gather(k)
            pltpu.make_async_copy(xs_hbm.at[src_v.at[pl.ds((k - 1) * CHUNK,
                                                           CHUNK)]],
                                  rows_v.at[k - 1], gsem).wait()
            scatter(k - 1)
        pltpu.make_async_copy(xs_hbm.at[src_v.at[pl.ds((KC - 1) * CHUNK,
                                                       CHUNK)]],
                              rows_v.at[KC - 1], gsem).wait()
        scatter(KC - 1)
        for k in range(KC):
            pltpu.make_async_copy(rows_v.at[k], acc_sh.at[dst_v.at[k]],
                                  ssem).wait()
        return carry

    lax.fori_loop(0, nblk, body, 0)
    plsc.subcore_barrier()

    pltpu.sync_copy(acc_sh.at[pl.ds(s * ROWS_PER_TILE, ROWS_PER_TILE)],
                    out_hbm.at[c, pl.ds(s * ROWS_PER_TILE, ROWS_PER_TILE)])


_sc_call = functools.partial(
    pl.kernel,
    out_type=jax.ShapeDtypeStruct((NC, N_PAD, W), jnp.float32),
    mesh=plsc.VectorSubcoreMesh(core_axis_name="c", subcore_axis_name="s"),
    scratch_types=[
        pltpu.VMEM((KC * CHUNK,), jnp.int32),
        pltpu.VMEM((KC, CHUNK), jnp.int32),
        pltpu.VMEM((KC, CHUNK, W), jnp.float32),
        pltpu.VMEM_SHARED((N_PAD, W), jnp.float32),
        pltpu.SemaphoreType.DMA,
        pltpu.SemaphoreType.DMA,
        pltpu.SemaphoreType.DMA,
    ],
    compiler_params=pltpu.CompilerParams(use_tc_tiling_on_sc=False),
)(_sc_body)


def kernel(standard_sample, edge_index, mean_param, diag_param, post_diag_param,
           alpha1, alpha2, gamma):
    f32 = jnp.float32
    zq = jnp.pad(standard_sample.astype(f32),
                 ((0, W - N_SAMPLES), (0, N_PAD - N_NODES)))
    dg = jnp.pad(diag_param, (0, N_PAD - N_NODES)).reshape(1, N_PAD)
    pd = jnp.pad(post_diag_param, (0, N_PAD - N_NODES)).reshape(1, N_PAD)
    mn = jnp.pad(mean_param, (0, N_PAD - N_NODES)).reshape(1, N_PAD)
    zeros_hbm = jnp.zeros((ZROWS, W), f32)
    scalars = jnp.stack([alpha1, alpha2, gamma]).astype(f32).reshape(1, 3)

    grid = (N_PAD // NBC,)
    blk_spec = pl.BlockSpec((W, NBC), lambda i: (0, i))
    par_spec = pl.BlockSpec((1, NBC), lambda i: (0, i))

    xs_packed = pl.pallas_call(
        _prep_body,
        grid=grid,
        in_specs=[blk_spec, par_spec, par_spec, par_spec],
        out_specs=blk_spec,
        out_shape=jax.ShapeDtypeStruct((W, N_PAD), f32),
    )(zq, dg, pd, mn)
    xs_rows = xs_packed.T

    acc = _sc_call(xs_rows, edge_index, zeros_hbm)

    RP = N_PAD * W // 128
    xs_pk = xs_rows.reshape(RP, 128)
    acc_pk = acc.reshape(NC, RP, 128)
    out_pk = pl.pallas_call(
        _final_body,
        grid=(RP // RB,),
        in_specs=[pl.BlockSpec(memory_space=pltpu.SMEM),
                  pl.BlockSpec((RB, 128), lambda i: (i, 0)),
                  pl.BlockSpec((NC, RB, 128), lambda i: (0, i, 0))],
        out_specs=pl.BlockSpec((RB, 128), lambda i: (i, 0)),
        out_shape=jax.ShapeDtypeStruct((RP, 128), f32),
    )(scalars, xs_pk, acc_pk)

    return out_pk.reshape(N_PAD, W)[:N_NODES, :N_SAMPLES].T


# speedup vs baseline: 1.0603x; 1.0603x over previous
import functools

import jax
import jax.numpy as jnp
from jax import lax
from jax.experimental import pallas as pl
from jax.experimental.pallas import tpu as pltpu
from jax.experimental.pallas import tpu_sc as plsc

N_NODES = 100000
N_SAMPLES = 10
N_EDGES = 1600000
W = 16
N_PAD = 100352
NC, NS = 2, 16
NW = NC * NS
CHUNK = 128
KC = 4
N_CHUNKS = N_EDGES // CHUNK
ROWS_PER_TILE = N_PAD // NS
ZROWS = 784
NBC = 2048


def _prep_body(z_ref, dg_ref, pd_ref, mn_ref, o_ref):
    z = z_ref[...]
    std = jax.nn.softplus(dg_ref[...])
    x = std * z
    row = lax.broadcasted_iota(jnp.int32, (W, NBC), 0)
    x = jnp.where(row == 10, 1.0, x)
    x = jnp.where(row == 11, jax.nn.softplus(pd_ref[...]), x)
    x = jnp.where(row == 12, mn_ref[...], x)
    o_ref[...] = x


def _final_body(s_ref, xs_ref, a_ref, o_ref):
    alpha1 = s_ref[0, 0]
    alpha2 = s_ref[0, 1]
    gamma = s_ref[0, 2]
    xs = xs_ref[...]
    a = a_ref[...]
    aggr = (a[0] + a[1]).T
    deg = jnp.maximum(aggr[10:11, :], 1.0)
    self_w = jnp.exp(gamma * jnp.log(deg))
    out = xs[11:12, :] * (alpha1 * self_w * xs + alpha2 * aggr) + xs[12:13, :]
    o_ref[...] = out


def _sc_body(xs_hbm, ei_hbm, zeros_hbm, out_hbm,
             src_v, dst_v, rows_v, acc_sh, gsem, ssem, dsem):
    c = lax.axis_index("c")
    s = lax.axis_index("s")
    base_blk = jnp.where(
        c == 0,
        jnp.where(s < 10, s * 99, 990 + (s - 10) * 98),
        1578 + jnp.where(s < 11, s * 97, 1067 + (s - 11) * 96))
    base_chunk = base_blk * KC
    nblk = jnp.where(c == 0, jnp.where(s < 10, 99, 98),
                     jnp.where(s < 11, 97, 96))

    for r in range(ROWS_PER_TILE // ZROWS):
        pltpu.sync_copy(zeros_hbm,
                        acc_sh.at[pl.ds(s * ROWS_PER_TILE + r * ZROWS, ZROWS)])
    plsc.subcore_barrier()

    def gather(k):
        return pltpu.async_copy(xs_hbm.at[src_v.at[pl.ds(k * CHUNK, CHUNK)]],
                                rows_v.at[k], gsem)

    def scatter(k):
        return pltpu.async_copy(rows_v.at[k], acc_sh.at[dst_v.at[k]], ssem,
                                add=True)

    def body(blk, carry):
        e0 = (base_chunk + blk * KC) * CHUNK
        pltpu.sync_copy(ei_hbm.at[0, pl.ds(e0, KC * CHUNK)], src_v)
        for k in range(KC):
            pltpu.async_copy(ei_hbm.at[1, pl.ds(e0 + k * CHUNK, CHUNK)],
                             dst_v.at[k], dsem)
        for k in range(KC):
            pltpu.make_async_copy(ei_hbm.at[1, pl.ds(e0 + k * CHUNK, CHUNK)],
                                  dst_v.at[k], dsem).wait()
        gather(0)
        for k in range(1, KC):
            gather(k)
            pltpu.make_async_copy(xs_hbm.at[src_v.at[pl.ds((k - 1) * CHUNK,
                                                           CHUNK)]],
                                  rows_v.at[k - 1], gsem).wait()
            scatter(k - 1)
        pltpu.make_async_copy(xs_hbm.at[src_v.at[pl.ds((KC - 1) * CHUNK,
                                                       CHUNK)]],
                              rows_v.at[KC - 1], gsem).wait()
        scatter(KC - 1)
        for k in range(KC):
            pltpu.make_async_copy(rows_v.at[k], acc_sh.at[dst_v.at[k]],
                                  ssem).wait()
        return carry

    lax.fori_loop(0, nblk, body, 0)
    plsc.subcore_barrier()

    pltpu.sync_copy(acc_sh.at[pl.ds(s * ROWS_PER_TILE, ROWS_PER_TILE)],
                    out_hbm.at[c, pl.ds(s * ROWS_PER_TILE, ROWS_PER_TILE)])


_sc_call = functools.partial(
    pl.kernel,
    out_type=jax.ShapeDtypeStruct((NC, N_PAD, W), jnp.float32),
    mesh=plsc.VectorSubcoreMesh(core_axis_name="c", subcore_axis_name="s"),
    scratch_types=[
        pltpu.VMEM((KC * CHUNK,), jnp.int32),
        pltpu.VMEM((KC, CHUNK), jnp.int32),
        pltpu.VMEM((KC, CHUNK, W), jnp.float32),
        pltpu.VMEM_SHARED((N_PAD, W), jnp.float32),
        pltpu.SemaphoreType.DMA,
        pltpu.SemaphoreType.DMA,
        pltpu.SemaphoreType.DMA,
    ],
    compiler_params=pltpu.CompilerParams(use_tc_tiling_on_sc=False),
)(_sc_body)


def kernel(standard_sample, edge_index, mean_param, diag_param, post_diag_param,
           alpha1, alpha2, gamma):
    f32 = jnp.float32
    zq = jnp.pad(standard_sample.astype(f32),
                 ((0, W - N_SAMPLES), (0, N_PAD - N_NODES)))
    dg = jnp.pad(diag_param, (0, N_PAD - N_NODES)).reshape(1, N_PAD)
    pd = jnp.pad(post_diag_param, (0, N_PAD - N_NODES)).reshape(1, N_PAD)
    mn = jnp.pad(mean_param, (0, N_PAD - N_NODES)).reshape(1, N_PAD)
    zeros_hbm = jnp.zeros((ZROWS, W), f32)
    scalars = jnp.stack([alpha1, alpha2, gamma]).astype(f32).reshape(1, 3)

    grid = (N_PAD // NBC,)
    blk_spec = pl.BlockSpec((W, NBC), lambda i: (0, i))
    par_spec = pl.BlockSpec((1, NBC), lambda i: (0, i))

    xs_packed = pl.pallas_call(
        _prep_body,
        grid=grid,
        in_specs=[blk_spec, par_spec, par_spec, par_spec],
        out_specs=blk_spec,
        out_shape=jax.ShapeDtypeStruct((W, N_PAD), f32),
    )(zq, dg, pd, mn)
    xs_rows = xs_packed.T

    acc = _sc_call(xs_rows, edge_index, zeros_hbm)

    out_packed = pl.pallas_call(
        _final_body,
        grid=grid,
        in_specs=[pl.BlockSpec(memory_space=pltpu.SMEM),
                  blk_spec,
                  pl.BlockSpec((NC, NBC, W), lambda i: (0, i, 0))],
        out_specs=blk_spec,
        out_shape=jax.ShapeDtypeStruct((W, N_PAD), f32),
    )(scalars, xs_packed, acc)

    return out_packed[:N_SAMPLES, :N_NODES]
